# baseline (speedup 1.0000x reference)
import functools

import jax
import jax.numpy as jnp
from jax import lax
from jax.experimental import pallas as pl
from jax.experimental.pallas import tpu as pltpu
from jax.experimental.pallas import tpu_sc as plsc

N_GENES = 100000
D = 64
B = 4096
L = 200
EPS = 1e-5

BL = B * L
NC = 2
NS = 16
NW = NC * NS
PW = BL // NW
CH = 128
NCHUNK = PW // CH
GR = 32


def _rsqrt_vec(x):
    i = lax.bitcast_convert_type(x, jnp.int32)
    i = jnp.int32(0x5F3759DF) - (i >> 1)
    y = lax.bitcast_convert_type(i, jnp.float32)
    for _ in range(3):
        y = y * (jnp.float32(1.5) - jnp.float32(0.5) * x * y * y)
    return y


def _sc_body(x_hbm, table_hbm, gs_hbm, bs_hbm, out_hbm,
             idx_v, rows_v, gs_v, bs_v, sem):
    wid = lax.axis_index("s") * NC + lax.axis_index("c")

    pltpu.sync_copy(x_hbm.at[wid], idx_v)
    pltpu.sync_copy(gs_hbm, gs_v)
    pltpu.sync_copy(bs_hbm, bs_v)

    iota = lax.iota(jnp.int32, 16)
    zero = jnp.zeros((16,), jnp.float32)
    inv_d = jnp.float32(1.0 / D)

    def chunk_body(c, carry):
        pltpu.async_copy(table_hbm.at[idx_v.at[c]], rows_v, sem).wait()

        def group_body(g, gcarry):
            ra = g * GR + iota
            rb = ra + 16
            sa = sb = qa = qb = zero
            for j in range(D):
                col = jnp.full((16,), j, jnp.int32)
                va = plsc.load_gather(rows_v, [ra, col])
                vb = plsc.load_gather(rows_v, [rb, col])
                sa = sa + va
                sb = sb + vb
                qa = qa + va * va
                qb = qb + vb * vb
            ma = sa * inv_d
            mb = sb * inv_d
            rsta = _rsqrt_vec(qa * inv_d - ma * ma + jnp.float32(EPS))
            rstb = _rsqrt_vec(qb * inv_d - mb * mb + jnp.float32(EPS))
            for j in range(D):
                col = jnp.full((16,), j, jnp.int32)
                gsp = gs_v[j, :]
                bsp = bs_v[j, :]
                va = plsc.load_gather(rows_v, [ra, col])
                vb = plsc.load_gather(rows_v, [rb, col])
                oa = (va - ma) * rsta * gsp + bsp
                ob = (vb - mb) * rstb * gsp + bsp
                plsc.store_scatter(rows_v, [ra, col], oa)
                plsc.store_scatter(rows_v, [rb, col], ob)
            return gcarry

        lax.fori_loop(0, CH // GR, group_body, 0)

        base = pl.multiple_of((wid * NCHUNK + c) * CH, CH)
        pltpu.sync_copy(rows_v, out_hbm.at[pl.ds(base, CH)])
        return carry

    lax.fori_loop(0, NCHUNK, chunk_body, 0)


@jax.jit
def kernel(x, table, gamma, beta):
    xw = x.astype(jnp.int32).reshape(NW, NCHUNK, CH)
    gs = jnp.broadcast_to(gamma.astype(jnp.float32)[:, None], (D, 16))
    bs = jnp.broadcast_to(beta.astype(jnp.float32)[:, None], (D, 16))
    mesh = plsc.VectorSubcoreMesh(core_axis_name="c", subcore_axis_name="s")
    run = functools.partial(
        pl.kernel,
        mesh=mesh,
        out_type=jax.ShapeDtypeStruct((BL, D), jnp.float32),
        scratch_types=[
            pltpu.VMEM((NCHUNK, CH), jnp.int32),
            pltpu.VMEM((CH, D), jnp.float32),
            pltpu.VMEM((D, 16), jnp.float32),
            pltpu.VMEM((D, 16), jnp.float32),
            pltpu.SemaphoreType.DMA,
        ],
        compiler_params=pltpu.CompilerParams(
            needs_layout_passes=False, use_tc_tiling_on_sc=False),
    )(_sc_body)
    out = run(xw, table, gs, bs)
    return out.reshape(B, L, D)


# speedup vs baseline: 3.3582x; 3.3582x over previous
import functools

import jax
import jax.numpy as jnp
from jax import lax
from jax.experimental import pallas as pl
from jax.experimental.pallas import tpu as pltpu
from jax.experimental.pallas import tpu_sc as plsc

N_GENES = 100000
D = 64
B = 4096
L = 200
EPS = 1e-5

BL = B * L
NC = 2
NS = 16
NW = NC * NS
PW = BL // NW
CH = 128
NCHUNK = PW // CH
UNROLL = 8


def _rsqrt_newton(x):
    i = lax.bitcast_convert_type(x, jnp.int32)
    i = jnp.int32(0x5F3759DF) - (i >> 1)
    y = lax.bitcast_convert_type(i, jnp.float32)
    for _ in range(3):
        y = y * (jnp.float32(1.5) - jnp.float32(0.5) * x * y * y)
    return y


def _sc_body(x_hbm, table_hbm, gamma_hbm, beta_hbm, out_hbm,
             idx_v, rows_v, g_v, b_v, sem):
    wid = lax.axis_index("s") * NC + lax.axis_index("c")

    pltpu.sync_copy(x_hbm.at[wid], idx_v)
    pltpu.sync_copy(gamma_hbm, g_v)
    pltpu.sync_copy(beta_hbm, b_v)
    g = [g_v[pl.ds(16 * k, 16)] for k in range(4)]
    bta = [b_v[pl.ds(16 * k, 16)] for k in range(4)]
    inv_d = jnp.float32(1.0 / D)

    def chunk_body(c, carry):
        pltpu.async_copy(table_hbm.at[idx_v.at[c]], rows_v, sem).wait()

        @plsc.parallel_loop(0, CH, unroll=UNROLL)
        def row_body(r):
            v = [rows_v[r, pl.ds(16 * k, 16)] for k in range(4)]
            s = (v[0] + v[1]) + (v[2] + v[3])
            s2 = (v[0] * v[0] + v[1] * v[1]) + (v[2] * v[2] + v[3] * v[3])
            mean = jnp.sum(s) * inv_d
            var = jnp.sum(s2) * inv_d - mean * mean
            rstd = _rsqrt_newton(var + jnp.float32(EPS))
            for k in range(4):
                rows_v[r, pl.ds(16 * k, 16)] = (v[k] - mean) * rstd * g[k] + bta[k]

        base = pl.multiple_of((wid * NCHUNK + c) * CH, CH)
        pltpu.sync_copy(rows_v, out_hbm.at[pl.ds(base, CH)])
        return carry

    lax.fori_loop(0, NCHUNK, chunk_body, 0)


@jax.jit
def kernel(x, table, gamma, beta):
    xw = x.astype(jnp.int32).reshape(NW, NCHUNK, CH)
    mesh = plsc.VectorSubcoreMesh(core_axis_name="c", subcore_axis_name="s")
    run = functools.partial(
        pl.kernel,
        mesh=mesh,
        out_type=jax.ShapeDtypeStruct((BL, D), jnp.float32),
        scratch_types=[
            pltpu.VMEM((NCHUNK, CH), jnp.int32),
            pltpu.VMEM((CH, D), jnp.float32),
            pltpu.VMEM((D,), jnp.float32),
            pltpu.VMEM((D,), jnp.float32),
            pltpu.SemaphoreType.DMA,
        ],
        compiler_params=pltpu.CompilerParams(
            needs_layout_passes=False, use_tc_tiling_on_sc=False),
    )(_sc_body)
    out = run(xw, table, gamma, beta)
    return out.reshape(B, L, D)
